# Initial kernel scaffold; baseline (speedup 1.0000x reference)
import jax
import jax.numpy as jnp
from jax.experimental import pallas as pl


def kernel(input, weight):
    raise NotImplementedError("write your pallas kernel here")


# speedup vs baseline: 1.1142x; 1.1142x over previous
import functools

import jax
import jax.numpy as jnp
from jax import lax
from jax.experimental import pallas as pl
from jax.experimental.pallas import tpu as pltpu
from jax.experimental.pallas import tpu_sc as plsc

NUM_ROWS = 1000000
DIM = 32
BATCH = 16384 * 50
CHUNK = 128
NCHUNKS = BATCH // CHUNK
NC, NS = 2, 16
NW = NC * NS
ROWS_PER_W = NCHUNKS // NW
NBUF = 8
NROUNDS = ROWS_PER_W // NBUF


def _embed_body(idx_hbm, table_hbm, out_hbm, idx_v, rows_v, gsem, ssem):
    wid = lax.axis_index("s") * NC + lax.axis_index("c")
    row0 = wid * ROWS_PER_W

    pltpu.sync_copy(idx_hbm.at[pl.ds(row0, ROWS_PER_W)], idx_v)

    def gather_start(j, b):
        pltpu.async_copy(table_hbm.at[idx_v.at[j]], rows_v.at[b], gsem.at[b])

    def gather_wait(b):
        pltpu.make_async_copy(
            table_hbm.at[idx_v.at[0]], rows_v.at[b], gsem.at[b]
        ).wait()

    def store_start(j, b):
        pltpu.async_copy(
            rows_v.at[b], out_hbm.at[pl.ds((row0 + j) * CHUNK, CHUNK)], ssem.at[b]
        )

    def store_wait(b):
        pltpu.make_async_copy(
            rows_v.at[b], out_hbm.at[pl.ds(row0 * CHUNK, CHUNK)], ssem.at[b]
        ).wait()

    for b in range(NBUF):
        gather_start(b, b)

    @pl.loop(0, NROUNDS)
    def _round(r):
        for b in range(NBUF):
            j = r * NBUF + b
            gather_wait(b)
            store_start(j, b)
            nxt = j + NBUF

            @pl.when(nxt < ROWS_PER_W)
            def _():
                store_wait(b)
                gather_start(nxt, b)

    for b in range(NBUF):
        store_wait(b)


@jax.jit
def _embed(idx2d, weight):
    mesh = plsc.VectorSubcoreMesh(core_axis_name="c", subcore_axis_name="s")
    run = pl.kernel(
        _embed_body,
        out_type=jax.ShapeDtypeStruct((BATCH, DIM), jnp.float32),
        mesh=mesh,
        compiler_params=pltpu.CompilerParams(use_tc_tiling_on_sc=False),
        scratch_types=[
            pltpu.VMEM((ROWS_PER_W, CHUNK), jnp.int32),
            pltpu.VMEM((NBUF, CHUNK, DIM), jnp.float32),
            pltpu.SemaphoreType.DMA((NBUF,)),
            pltpu.SemaphoreType.DMA((NBUF,)),
        ],
    )
    return run(idx2d, weight)


def kernel(input, weight):
    idx2d = input.reshape(NCHUNKS, CHUNK).astype(jnp.int32)
    out = _embed(idx2d, weight)
    return out.reshape(input.shape[0], input.shape[1], DIM)


# speedup vs baseline: 1.7924x; 1.6087x over previous
import jax
import jax.numpy as jnp
from jax import lax
from jax.experimental import pallas as pl
from jax.experimental.pallas import tpu as pltpu
from jax.experimental.pallas import tpu_sc as plsc

NUM_ROWS = 1000000
DIM = 32
SEQ = 16384
SLEN = 50
NC, NS = 2, 16
NW = NC * NS
SENT_PER_W = SEQ // NW
NBUF = 8
NROUNDS = SENT_PER_W // NBUF


def _embed_body(idx_hbm, table_hbm, out_hbm, idx_v, rows_v, gsem, ssem):
    wid = lax.axis_index("s") * NC + lax.axis_index("c")
    s0 = wid * SENT_PER_W

    pltpu.sync_copy(idx_hbm.at[pl.ds(s0, SENT_PER_W)], idx_v)

    def gather_start(k, b):
        pltpu.async_copy(table_hbm.at[idx_v.at[k]], rows_v.at[b], gsem.at[b])

    def gather_wait(b):
        pltpu.make_async_copy(
            table_hbm.at[idx_v.at[0]], rows_v.at[b], gsem.at[b]
        ).wait()

    def store_start(k, b):
        pltpu.async_copy(rows_v.at[b], out_hbm.at[s0 + k], ssem.at[b])

    def store_wait(b):
        pltpu.make_async_copy(rows_v.at[b], out_hbm.at[s0], ssem.at[b]).wait()

    for b in range(NBUF):
        gather_start(b, b)

    @pl.loop(0, NROUNDS)
    def _round(r):
        for b in range(NBUF):
            k = r * NBUF + b
            gather_wait(b)
            store_start(k, b)
            nxt = k + NBUF

            @pl.when(nxt < SENT_PER_W)
            def _():
                store_wait(b)
                gather_start(nxt, b)

    for b in range(NBUF):
        store_wait(b)


@jax.jit
def _embed(idx, weight):
    mesh = plsc.VectorSubcoreMesh(core_axis_name="c", subcore_axis_name="s")
    run = pl.kernel(
        _embed_body,
        out_type=jax.ShapeDtypeStruct((SEQ, SLEN, DIM), jnp.float32),
        mesh=mesh,
        compiler_params=pltpu.CompilerParams(use_tc_tiling_on_sc=False),
        scratch_types=[
            pltpu.VMEM((SENT_PER_W, SLEN), jnp.int32),
            pltpu.VMEM((NBUF, SLEN, DIM), jnp.float32),
            pltpu.SemaphoreType.DMA((NBUF,)),
            pltpu.SemaphoreType.DMA((NBUF,)),
        ],
    )
    return run(idx, weight)


def kernel(input, weight):
    return _embed(input.astype(jnp.int32), weight)
